# baseline (speedup 1.0000x reference)
import jax
import jax.numpy as jnp
from jax.experimental import pallas as pl

N = 10000
HEADS = 8
G = 64


def _identity_kernel(x_ref, o_ref):
    o_ref[...] = x_ref[...]


def _gcn(x, W, b, src, dst, norm):
    h = x @ W
    msg = h[src] * norm[:, None]
    return jax.ops.segment_sum(msg, dst, num_segments=N) + b


def kernel(x, edge_index, batch, W1, b1, W2, b2, W3, b3, W4, b4, Wg, att_src, att_dst, bg, Wl1, bl1, Wl2, bl2):
    loop = jnp.arange(N)
    src = jnp.concatenate([edge_index[0], loop])
    dst = jnp.concatenate([edge_index[1], loop])
    deg = jnp.zeros((N,), jnp.float32).at[dst].add(1.0)
    dinv = jnp.where(deg > 0, 1.0 / jnp.sqrt(deg), 0.0)
    norm = dinv[src] * dinv[dst]

    h = jax.nn.relu(_gcn(x, W1, b1, src, dst, norm))
    h = jax.nn.relu(_gcn(h, W2, b2, src, dst, norm))
    h = jax.nn.relu(_gcn(h, W3, b3, src, dst, norm))
    h = jax.nn.relu(_gcn(h, W4, b4, src, dst, norm))

    HID = h.shape[1]
    hh = (h @ Wg).reshape(N, HEADS, HID)
    a_s = (hh * att_src[None, :, :]).sum(-1)
    a_d = (hh * att_dst[None, :, :]).sum(-1)
    e = jax.nn.leaky_relu(a_s[src] + a_d[dst], negative_slope=0.2)
    emax = jax.ops.segment_max(e, dst, num_segments=N)
    emax = jnp.where(jnp.isfinite(emax), emax, 0.0)
    ex = jnp.exp(e - emax[dst])
    den = jax.ops.segment_sum(ex, dst, num_segments=N)
    alpha = ex / (den[dst] + 1e-16)
    gat = jax.ops.segment_sum(hh[src] * alpha[:, :, None], dst, num_segments=N).mean(axis=1) + bg
    h = jax.nn.relu(gat)

    cnt = jax.ops.segment_sum(jnp.ones((N,), jnp.float32), batch, num_segments=G)
    mean_p = jax.ops.segment_sum(h, batch, num_segments=G) / jnp.maximum(cnt, 1.0)[:, None]
    max_p = jax.ops.segment_max(h, batch, num_segments=G)
    max_p = jnp.where(jnp.isfinite(max_p), max_p, 0.0)
    p = mean_p + max_p

    p = jax.nn.relu(p @ Wl1 + bl1)
    out = p @ Wl2 + bl2
    out = pl.pallas_call(
        _identity_kernel,
        out_shape=jax.ShapeDtypeStruct(out.shape, out.dtype),
    )(out)
    return out


# speedup vs baseline: 2.4028x; 2.4028x over previous
import functools
import jax
import jax.numpy as jnp
from jax import lax
from jax.experimental import pallas as pl
from jax.experimental.pallas import tpu as pltpu
from jax.experimental.pallas import tpu_sc as plsc

N = 10000
HID = 512
HEADS = 8
G = 64
NC, NS, L = 2, 16, 16
NW = NC * NS
NDW = 320
NP = NW * NDW
KCH = 32

_mesh = plsc.VectorSubcoreMesh(core_axis_name="c", subcore_axis_name="s")


def _wid():
    return lax.axis_index("s") * NC + lax.axis_index("c")


def _recip16(v):
    sv = v
    rv = jnp.full((L,), 1.0, jnp.float32)
    for _ in range(14):
        st = jnp.clip((sv - 2.0) * 1e30, 0.0, 1.0)
        f = 1.0 - 0.5 * st
        sv = sv * f
        rv = rv * f
    y = 1.4571 - 0.4571 * sv
    for _ in range(3):
        y = y * (2.0 - sv * y)
    return rv * y


def _mm_body(x_ref, w_ref, o_ref):
    o_ref[...] = jnp.dot(x_ref[...], w_ref[...],
                         preferred_element_type=jnp.float32)


def _mm(x, W, bm=512):
    M, K = x.shape
    NO = W.shape[1]
    return pl.pallas_call(
        _mm_body,
        grid=(M // bm,),
        in_specs=[pl.BlockSpec((bm, K), lambda i: (i, 0)),
                  pl.BlockSpec((K, NO), lambda i: (0, 0))],
        out_specs=pl.BlockSpec((bm, NO), lambda i: (i, 0)),
        out_shape=jax.ShapeDtypeStruct((M, NO), jnp.float32),
    )(x, W)


def _mms_body(x_ref, w_ref, s_ref, o_ref):
    o_ref[...] = jnp.dot(x_ref[...], w_ref[...],
                         preferred_element_type=jnp.float32) * s_ref[...]


def _mm_scaled(x, W, scale, bm=512):
    M, K = x.shape
    NO = W.shape[1]
    return pl.pallas_call(
        _mms_body,
        grid=(M // bm,),
        in_specs=[pl.BlockSpec((bm, K), lambda i: (i, 0)),
                  pl.BlockSpec((K, NO), lambda i: (0, 0)),
                  pl.BlockSpec((bm, 1), lambda i: (i, 0))],
        out_specs=pl.BlockSpec((bm, NO), lambda i: (i, 0)),
        out_shape=jax.ShapeDtypeStruct((M, NO), jnp.float32),
    )(x, W, scale)


def _mlp_body(p_ref, w1_ref, b1_ref, w2_ref, b2_ref, o_ref):
    t = jnp.maximum(
        jnp.dot(p_ref[...], w1_ref[...], preferred_element_type=jnp.float32)
        + b1_ref[...], 0.0)
    o_ref[...] = jnp.dot(t, w2_ref[...],
                         preferred_element_type=jnp.float32) + b2_ref[...]


def _mlp(p, W1, b1, W2p, b2p):
    return pl.pallas_call(
        _mlp_body,
        out_shape=jax.ShapeDtypeStruct((G, 128), jnp.float32),
    )(p, W1, b1, W2p, b2p)


@functools.cache
def _build_k_deg():
    return pl.kernel(
        _k_deg_body,
        out_type=jax.ShapeDtypeStruct((NW * NDW * L,), jnp.float32),
        mesh=_mesh,
        scratch_types=[pltpu.VMEM((L,), jnp.float32),
                       pltpu.VMEM((L,), jnp.int32),
                       pltpu.VMEM((NDW * L,), jnp.float32)],
    )


def _k_deg_body(dst_hbm, meta_hbm, deg_hbm, metav, dstv, tab_v):
    w = _wid()
    pltpu.sync_copy(meta_hbm.at[w], metav)
    mv = metav[pl.ds(0, L)]
    e0 = mv[0].astype(jnp.int32)
    e1 = mv[1].astype(jnp.int32)
    d0 = w * NDW
    z = jnp.zeros((L,), jnp.float32)
    lane0f = jnp.maximum(
        1.0 - lax.iota(jnp.int32, L).astype(jnp.float32), 0.0)

    def zi(i, _):
        tab_v[pl.ds(i * L, L)] = z
        return ()
    lax.fori_loop(0, NDW, zi, ())

    a0 = (e0 // L) * L
    nch = (e1 - a0 + (L - 1)) // L

    def chunk(c, _):
        base = a0 + c * L
        pltpu.sync_copy(dst_hbm.at[pl.ds(base, L)], dstv)
        dv = dstv[pl.ds(0, L)]
        for j in range(L):
            valid = jnp.logical_and((base + j) >= e0, (base + j) < e1)
            vf = valid.astype(jnp.float32)
            q = jnp.clip(dv[j] - d0, 0, NDW - 1) * L
            tab_v[pl.ds(q, L)] = tab_v[pl.ds(q, L)] + lane0f * vf
        return ()
    lax.fori_loop(0, nch, chunk, ())

    pltpu.sync_copy(tab_v, deg_hbm.at[pl.ds(w * NDW * L, NDW * L)])


@functools.cache
def _build_k_gcn():
    return pl.kernel(
        _k_gcn_body,
        out_type=jax.ShapeDtypeStruct((NP, HID), jnp.float32),
        mesh=_mesh,
        scratch_types=[pltpu.VMEM((L,), jnp.float32),
                       pltpu.VMEM((L,), jnp.int32),
                       pltpu.VMEM((L,), jnp.int32),
                       pltpu.VMEM((NDW + L,), jnp.float32),
                       pltpu.VMEM((L, HID), jnp.float32),
                       pltpu.VMEM((HID,), jnp.float32),
                       pltpu.VMEM((HID,), jnp.float32),
                       pltpu.VMEM((HID,), jnp.float32),
                       pltpu.SemaphoreType.DMA],
    )


def _k_gcn_body(h_hbm, src_hbm, dst_hbm, meta_hbm, dinv_hbm, b_hbm, out_hbm,
                metav, srcv, dstv, dinvw, gbuf, accv, tmpv, bv, sem):
    w = _wid()
    pltpu.sync_copy(meta_hbm.at[w], metav)
    pltpu.sync_copy(b_hbm, bv)
    d0 = w * NDW
    pltpu.sync_copy(dinv_hbm.at[pl.ds(d0, NDW)], dinvw.at[pl.ds(0, NDW)])
    mv = metav[pl.ds(0, L)]
    e0 = mv[0].astype(jnp.int32)
    e1 = mv[1].astype(jnp.int32)
    dprev0 = mv[2].astype(jnp.int32)
    z = jnp.zeros((L,), jnp.float32)

    def zi(i, _):
        accv[pl.ds(i * L, L)] = z
        return ()
    lax.fori_loop(0, KCH, zi, ())

    def flush(dp):
        sval = dinvw[pl.ds(dp - d0, L)][0]
        sb = jnp.full((L,), sval, jnp.float32)

        def fk(k, _):
            tmpv[pl.ds(k * L, L)] = jnp.maximum(
                accv[pl.ds(k * L, L)] * sb + bv[pl.ds(k * L, L)], 0.0)
            return ()
        lax.fori_loop(0, KCH, fk, ())
        pltpu.sync_copy(tmpv, out_hbm.at[dp])

    a0 = (e0 // L) * L
    nch = (e1 - a0 + (L - 1)) // L

    def chunk(c, dprev):
        base = a0 + c * L
        pltpu.sync_copy(src_hbm.at[pl.ds(base, L)], srcv)
        pltpu.sync_copy(dst_hbm.at[pl.ds(base, L)], dstv)
        pltpu.async_copy(h_hbm.at[srcv], gbuf, sem).wait()
        dv = dstv[pl.ds(0, L)]
        for j in range(L):
            d = dv[j]
            valid = jnp.logical_and((base + j) >= e0, (base + j) < e1)
            c_fl = jnp.logical_and(valid, d != dprev)
            c_ac = jnp.logical_and(valid, d == dprev)

            def fl(dp):
                flush(dp)

                def rk(k, _):
                    accv[pl.ds(k * L, L)] = gbuf[j, pl.ds(k * L, L)]
                    return ()
                lax.fori_loop(0, KCH, rk, ())
                return ()

            lax.cond(c_fl, fl, lambda dp: (), dprev)

            def ac(_):
                def rk(k, _):
                    accv[pl.ds(k * L, L)] = (accv[pl.ds(k * L, L)]
                                             + gbuf[j, pl.ds(k * L, L)])
                    return ()
                lax.fori_loop(0, KCH, rk, ())
                return ()

            lax.cond(c_ac, ac, lambda _: (), 0)
            dprev = jnp.where(valid, d, dprev)
        return dprev

    dlast = lax.fori_loop(0, nch, chunk, dprev0)
    flush(dlast)


@functools.cache
def _build_k_gat():
    return pl.kernel(
        _k_gat_body,
        out_type=jax.ShapeDtypeStruct((NP, HID), jnp.float32),
        mesh=_mesh,
        scratch_types=[pltpu.VMEM((L,), jnp.float32),
                       pltpu.VMEM((L,), jnp.int32),
                       pltpu.VMEM((L,), jnp.int32),
                       pltpu.VMEM((L, HEADS * HID), jnp.float32),
                       pltpu.VMEM((L, 128), jnp.float32),
                       pltpu.VMEM((L, 128), jnp.float32),
                       pltpu.VMEM((HEADS * HID,), jnp.float32),
                       pltpu.VMEM((HID,), jnp.float32),
                       pltpu.VMEM((HID,), jnp.float32),
                       pltpu.VMEM((L,), jnp.float32),
                       pltpu.VMEM((L,), jnp.float32),
                       pltpu.SemaphoreType.DMA],
    )


def _k_gat_body(hh_hbm, ae_hbm, src_hbm, dst_hbm, meta_hbm, bg_hbm, out_hbm,
                metav, srcv, dstv, gbuf, gs, gd, accv, tmpv, bgv,
                msv, ssv, sem):
    w = _wid()
    pltpu.sync_copy(meta_hbm.at[w], metav)
    pltpu.sync_copy(bg_hbm, bgv)
    mv = metav[pl.ds(0, L)]
    e0 = mv[0].astype(jnp.int32)
    e1 = mv[1].astype(jnp.int32)
    dprev0 = mv[2].astype(jnp.int32)
    z = jnp.zeros((L,), jnp.float32)
    hmf = jnp.clip(
        jnp.float32(HEADS) - lax.iota(jnp.int32, L).astype(jnp.float32),
        0.0, 1.0)

    def zi(i, _):
        accv[pl.ds(i * L, L)] = z
        return ()
    lax.fori_loop(0, HEADS * KCH, zi, ())

    def gat_flush(dp, s_):
        inv16 = _recip16(jnp.maximum(s_, 1.0))

        def fk(k, _):
            o = z
            for h in range(HEADS):
                o = o + accv[pl.ds(h * HID + k * L, L)] * \
                    jnp.full((L,), inv16[h], jnp.float32)
            tmpv[pl.ds(k * L, L)] = jnp.maximum(
                o * (1.0 / HEADS) + bgv[pl.ds(k * L, L)], 0.0)
            return ()
        lax.fori_loop(0, KCH, fk, ())
        pltpu.sync_copy(tmpv, out_hbm.at[dp])

    msv[pl.ds(0, L)] = jnp.full((L,), -1e30, jnp.float32)
    ssv[pl.ds(0, L)] = z
    a0 = (e0 // L) * L
    nch = (e1 - a0 + (L - 1)) // L

    def chunk(c, dprev):
        base = a0 + c * L
        pltpu.sync_copy(src_hbm.at[pl.ds(base, L)], srcv)
        pltpu.sync_copy(dst_hbm.at[pl.ds(base, L)], dstv)
        pltpu.async_copy(hh_hbm.at[srcv], gbuf, sem).wait()
        pltpu.async_copy(ae_hbm.at[srcv], gs, sem).wait()
        pltpu.async_copy(ae_hbm.at[dstv], gd, sem).wait()
        dv = dstv[pl.ds(0, L)]
        for j in range(L):
            d = dv[j]
            s16 = gs[j, pl.ds(0, L)]
            d16 = gd[j, pl.ds(L, L)]
            xja = s16 + d16
            e = jnp.maximum(xja, 0.2 * xja)
            e = e * hmf + (-1e30) * (1.0 - hmf)
            valid = jnp.logical_and((base + j) >= e0, (base + j) < e1)
            c_new = jnp.logical_and(valid, d != dprev)
            c_same = jnp.logical_and(valid, d == dprev)

            def fl(_):
                gat_flush(dprev, ssv[pl.ds(0, L)])

                def rk(i, _):
                    accv[pl.ds(i * L, L)] = gbuf[j, pl.ds(i * L, L)]
                    return ()
                lax.fori_loop(0, HEADS * KCH, rk, ())
                msv[pl.ds(0, L)] = e
                ssv[pl.ds(0, L)] = jnp.full((L,), 1.0, jnp.float32)
                return ()

            lax.cond(c_new, fl, lambda _: (), 0)

            def upd(_):
                m_ = msv[pl.ds(0, L)]
                s_ = ssv[pl.ds(0, L)]
                mn = jnp.maximum(m_, e)
                scl = jnp.exp(m_ - mn)
                exl = jnp.exp(e - mn)
                for h in range(HEADS):
                    sch = jnp.full((L,), scl[h], jnp.float32)
                    exh = jnp.full((L,), exl[h], jnp.float32)

                    def kloop(k, _):
                        sl = pl.ds(h * HID + k * L, L)
                        accv[sl] = accv[sl] * sch + gbuf[j, sl] * exh
                        return ()
                    lax.fori_loop(0, KCH, kloop, ())
                msv[pl.ds(0, L)] = mn
                ssv[pl.ds(0, L)] = s_ * scl + exl
                return ()

            lax.cond(c_same, upd, lambda _: (), 0)
            dprev = jnp.where(valid, d, dprev)
        return dprev

    dlast = lax.fori_loop(0, nch, chunk, dprev0)
    gat_flush(dlast, ssv[pl.ds(0, L)])


@functools.cache
def _build_k_pool():
    return pl.kernel(
        _k_pool_body,
        out_type=jax.ShapeDtypeStruct((G, HID), jnp.float32),
        mesh=_mesh,
        scratch_types=[pltpu.VMEM((L,), jnp.float32),
                       pltpu.VMEM((L, HID), jnp.float32),
                       pltpu.VMEM((HID,), jnp.float32),
                       pltpu.VMEM((HID,), jnp.float32),
                       pltpu.VMEM((HID,), jnp.float32)],
    )


def _k_pool_body(h_hbm, meta_hbm, out_hbm, metav, rbuf, sumv, maxv, tmpv):
    w = _wid()
    pltpu.sync_copy(meta_hbm.at[w], metav)
    mv = metav[pl.ds(0, L)]
    z = jnp.zeros((L,), jnp.float32)
    neg = jnp.full((L,), -1e30, jnp.float32)
    for t in range(2):
        g = w * 2 + t
        b0 = mv[t].astype(jnp.int32)
        b1 = mv[t + 1].astype(jnp.int32)

        def zi(i, _):
            sumv[pl.ds(i * L, L)] = z
            maxv[pl.ds(i * L, L)] = neg
            return ()
        lax.fori_loop(0, KCH, zi, ())

        a0 = (b0 // L) * L
        nch = (b1 - a0 + (L - 1)) // L

        def chunk(c, _):
            base = a0 + c * L
            pltpu.sync_copy(h_hbm.at[pl.ds(base, L)], rbuf)
            for j in range(L):
                inb = jnp.logical_and((base + j) >= b0, (base + j) < b1)
                vf = inb.astype(jnp.float32)
                nvf = 1.0 - vf

                def rk(k, _):
                    r = rbuf[j, pl.ds(k * L, L)]
                    sumv[pl.ds(k * L, L)] = sumv[pl.ds(k * L, L)] + r * vf
                    maxv[pl.ds(k * L, L)] = jnp.maximum(
                        maxv[pl.ds(k * L, L)], r * vf + (-1e30) * nvf)
                    return ()
                lax.fori_loop(0, KCH, rk, ())
            return ()
        lax.fori_loop(0, nch, chunk, ())

        inv = jnp.full((L,), mv[3 + t], jnp.float32)
        cg = jnp.full((L,), mv[5 + t], jnp.float32)

        def wk(k, _):
            tmpv[pl.ds(k * L, L)] = (sumv[pl.ds(k * L, L)] * inv
                                     + maxv[pl.ds(k * L, L)] * cg)
            return ()
        lax.fori_loop(0, KCH, wk, ())
        pltpu.sync_copy(tmpv, out_hbm.at[g])


def kernel(x, edge_index, batch, W1, b1, W2, b2, W3, b3, W4, b4,
           Wg, att_src, att_dst, bg, Wl1, bl1, Wl2, bl2):
    loop = jnp.arange(N, dtype=jnp.int32)
    src_all = jnp.concatenate([edge_index[0].astype(jnp.int32), loop])
    dst_all = jnp.concatenate([edge_index[1].astype(jnp.int32), loop])
    order = jnp.argsort(dst_all)
    dst_s = dst_all[order]
    src_s = src_all[order]
    padz = jnp.zeros((L,), jnp.int32)
    dst_sp = jnp.concatenate([dst_s, padz])
    src_sp = jnp.concatenate([src_s, padz])
    erp = jnp.searchsorted(dst_s, jnp.arange(NW + 1, dtype=jnp.int32) * NDW
                           ).astype(jnp.int32)
    fd = dst_s[jnp.minimum(erp[:NW], dst_s.shape[0] - 1)]
    em = jnp.stack([erp[:NW], erp[1:NW + 1], fd], axis=1).astype(jnp.float32)
    em = jnp.pad(em, ((0, 0), (0, L - 3)))
    brp = jnp.searchsorted(batch.astype(jnp.int32),
                           jnp.arange(G + 1, dtype=jnp.int32)
                           ).astype(jnp.int32)
    cnts = (brp[1:] - brp[:-1]).astype(jnp.float32)
    invc = 1.0 / jnp.maximum(cnts, 1.0)
    cgf = (cnts > 0).astype(jnp.float32)
    pm = jnp.stack([brp[0:G:2].astype(jnp.float32),
                    brp[1:G + 1:2].astype(jnp.float32),
                    brp[2:G + 2:2].astype(jnp.float32),
                    invc[0::2], invc[1::2], cgf[0::2], cgf[1::2]],
                   axis=1)
    pm = jnp.pad(pm, ((0, 0), (0, L - 7)))

    deg = _build_k_deg()(dst_sp, em).reshape(NP, L)[:, 0]
    dinv = lax.rsqrt(jnp.maximum(deg, 1.0)).reshape(NP, 1)

    k_gcn = _build_k_gcn()
    xp = jnp.pad(x, ((0, NP - N), (0, 0)))
    h = k_gcn(_mm_scaled(xp, W1, dinv), src_sp, dst_sp, em, dinv[:, 0], b1)
    h = k_gcn(_mm_scaled(h, W2, dinv), src_sp, dst_sp, em, dinv[:, 0], b2)
    h = k_gcn(_mm_scaled(h, W3, dinv), src_sp, dst_sp, em, dinv[:, 0], b3)
    h = k_gcn(_mm_scaled(h, W4, dinv), src_sp, dst_sp, em, dinv[:, 0], b4)

    hh = _mm(h, Wg)
    A = jnp.zeros((HEADS * HID, 128), jnp.float32)
    for hd in range(HEADS):
        A = A.at[hd * HID:(hd + 1) * HID, hd].set(att_src[hd])
        A = A.at[hd * HID:(hd + 1) * HID, L + hd].set(att_dst[hd])
    ae = _mm(hh, A)

    h5 = _build_k_gat()(hh, ae, src_sp, dst_sp, em, bg)
    p = _build_k_pool()(h5, pm)

    W2p = jnp.pad(Wl2, ((0, 0), (0, 128 - Wl2.shape[1])))
    b2p = jnp.pad(bl2, (0, 128 - bl2.shape[0]))
    out = _mlp(p, Wl1, bl1.reshape(1, -1), W2p, b2p.reshape(1, -1))
    return out[:, :Wl2.shape[1]]
